# Initial kernel scaffold; baseline (speedup 1.0000x reference)
import jax
import jax.numpy as jnp
from jax.experimental import pallas as pl


def kernel(pc, params):
    raise NotImplementedError("write your pallas kernel here")


# speedup vs baseline: 3.6275x; 3.6275x over previous
import functools

import jax
import jax.numpy as jnp
from jax import lax
from jax.experimental import pallas as pl
from jax.experimental.pallas import tpu as pltpu
from jax.experimental.pallas import tpu_sc as plsc

EPS = 1e-5
_NUM_SC_WORKERS = 32
_SC_CHUNK = 128


def _geom_body(N, S, K, bw, xyzT_ref, xyz_ref, invd_ref, nx_ref, idx_ref,
               sq_ref, d_ref):
    b = pl.program_id(0)
    xt = xyzT_ref[0]
    x = xyz_ref[0]
    g = lax.dot_general(xt, x, (((1,), (0,)), ((), ())),
                        preferred_element_type=jnp.float32)
    n2c = jnp.sum(xt * xt, axis=1, keepdims=True)
    n2r = jnp.sum(x * x, axis=0, keepdims=True)
    sq = -2.0 * g + n2c + n2r
    sq_ref[...] = sq
    gg = jnp.exp(-sq / (2.0 * bw * bw)) / (2.5 * bw)
    invd_ref[0] = 1.0 / jnp.mean(gg, axis=1, keepdims=True)

    def fps_body(i, carry):
        dist, far = carry
        c = xyzT_ref[0, pl.ds(far, 1), :]
        nx_ref[0, pl.ds(i, 1), :] = c
        d_ref[pl.ds(i, 1), :] = sq_ref[pl.ds(far, 1), :]
        d = ((x[0:1, :] - c[:, 0:1]) ** 2 + (x[1:2, :] - c[:, 1:2]) ** 2
             + (x[2:3, :] - c[:, 2:3]) ** 2)
        dist = jnp.minimum(dist, d)
        nfar = jnp.argmax(dist).astype(jnp.int32)
        return dist, nfar

    lax.fori_loop(0, S, fps_body,
                  (jnp.full((1, N), 1e10, jnp.float32), jnp.int32(0)))

    off = b * N
    lane = lax.broadcasted_iota(jnp.int32, (S, N), 1)
    for k in range(K):
        dm = d_ref[...]
        sel = jnp.argmin(dm, axis=1).astype(jnp.int32)
        idx_ref[0, :, k:k + 1] = (sel + off)[:, None]
        d_ref[...] = jnp.where(lane == sel[:, None], jnp.float32(jnp.inf), dm)


def _geom(xyz, xyzT, S, K, bw):
    B, _, N = xyz.shape
    body = functools.partial(_geom_body, N, S, K, bw)
    return pl.pallas_call(
        body,
        grid=(B,),
        in_specs=[pl.BlockSpec((1, N, 3), lambda b: (b, 0, 0)),
                  pl.BlockSpec((1, 3, N), lambda b: (b, 0, 0))],
        out_specs=[pl.BlockSpec((1, N, 1), lambda b: (b, 0, 0)),
                   pl.BlockSpec((1, S, 3), lambda b: (b, 0, 0)),
                   pl.BlockSpec((1, S, K), lambda b: (b, 0, 0))],
        out_shape=[jax.ShapeDtypeStruct((B, N, 1), jnp.float32),
                   jax.ShapeDtypeStruct((B, S, 3), jnp.float32),
                   jax.ShapeDtypeStruct((B, S, K), jnp.int32)],
        scratch_shapes=[pltpu.VMEM((N, N), jnp.float32),
                        pltpu.VMEM((S, N), jnp.float32)],
    )(xyzT, xyz)


def _gather_rows(table, idx):
    v, d = table.shape
    bp = idx.shape[0]
    bpw = bp // _NUM_SC_WORKERS
    nloops = bpw // _SC_CHUNK
    assert bpw % _SC_CHUNK == 0

    mesh = plsc.VectorSubcoreMesh(core_axis_name="c", subcore_axis_name="s")

    @functools.partial(
        pl.kernel, mesh=mesh,
        out_type=jax.ShapeDtypeStruct((bp, d), jnp.float32),
        scratch_types=[pltpu.VMEM((_SC_CHUNK,), jnp.int32),
                       pltpu.VMEM((_SC_CHUNK, d), jnp.float32),
                       pltpu.SemaphoreType.DMA],
        compiler_params=pltpu.CompilerParams(use_tc_tiling_on_sc=False),
    )
    def sc_gather(table_hbm, idx_hbm, out_hbm, idx_v, rows_v, sem):
        wid = lax.axis_index("s") * 2 + lax.axis_index("c")
        base = wid * bpw

        def loop(i, _):
            off = base + i * _SC_CHUNK
            pltpu.sync_copy(idx_hbm.at[pl.ds(off, _SC_CHUNK)], idx_v)
            pltpu.async_copy(table_hbm.at[idx_v], rows_v, sem).wait()
            pltpu.sync_copy(rows_v, out_hbm.at[pl.ds(off, _SC_CHUNK)])
            return 0

        lax.fori_loop(0, nloops, loop, 0)

    return sc_gather(table, idx)


def _conv(x, wt_ref, b_ref):
    bs, k, ci = x.shape
    co = wt_ref.shape[1]
    xf = x.reshape(bs * k, ci)
    y = lax.dot_general(xf.astype(jnp.bfloat16),
                        wt_ref[...].astype(jnp.bfloat16),
                        (((1,), (0,)), ((), ())),
                        preferred_element_type=jnp.float32)
    return y.reshape(bs, k, co) + b_ref[...][None]


def _bn3(x, g_ref, be_ref):
    bs, k, c = x.shape
    xf = x.reshape(bs * k, c)
    m = jnp.mean(xf, axis=0)
    v = jnp.mean((xf - m) ** 2, axis=0)
    y = (x - m[None, None]) / jnp.sqrt(v[None, None] + EPS)
    return y * g_ref[...][None] + be_ref[...][None]


def _sa_tail_body(BS, K, C, Co, pt_lo, dc, cmax, refs):
    (g_ref, nx_ref,
     mw, mb, mg, mbe,
     w1, wb1, wg1, wbe1, w2, wb2, wg2, wbe2, w3, wb3, wg3, wbe3,
     d1, db1, dg1, dbe1, d2, db2, dg2, dbe2, d3, db3, dg3, dbe3,
     lp, lb, blg, blbe, out_ref, st, ys) = refs
    ph = pl.program_id(0)
    c = pl.program_id(1)
    cb = out_ref.shape[0]
    pcount = float(BS * K)

    def acc(slot, x3):
        cc = x3.shape[2]
        xf = x3.reshape(x3.shape[0] * K, cc)
        st[2 * slot, 0:cc] += jnp.sum(xf, axis=0)
        st[2 * slot + 1, 0:cc] += jnp.sum(xf * xf, axis=0)

    def bn(slot, x3, g, be, n=pcount):
        cc = x3.shape[2]
        m = st[2 * slot, 0:cc] / n
        v = st[2 * slot + 1, 0:cc] / n - m * m
        y = (x3 - m[None, None]) / jnp.sqrt(v[None, None] + EPS)
        return y * g[...][None] + be[...][None]

    def gx_of(gm):
        return gm[:, :, 0:3] - nx_ref[...]

    def wn_pre(gm, upto):
        h = _conv(gx_of(gm), w1, wb1)
        if upto >= 2:
            h = _conv(jnp.maximum(bn(1, h, wg1, wbe1), 0.0), w2, wb2)
        if upto >= 3:
            h = _conv(jnp.maximum(bn(2, h, wg2, wbe2), 0.0), w3, wb3)
        return h

    def dn_pre(gm, upto):
        dcol = gm[:, :, dc:dc + 1]
        ratio = dcol / jnp.max(dcol, axis=1, keepdims=True)
        h = _conv(ratio, d1, db1)
        if upto >= 2:
            h = _conv(jnp.maximum(bn(4, h, dg1, dbe1), 0.0), d2, db2)
        if upto >= 3:
            h = _conv(jnp.maximum(bn(5, h, dg2, dbe2), 0.0), d3, db3)
        return h

    def x_pre(gm):
        npts = jnp.concatenate([gx_of(gm), gm[:, :, pt_lo:pt_lo + C]], axis=2)
        return _conv(npts, mw, mb)

    @pl.when(jnp.logical_and(ph == 0, c == 0))
    def _():
        st[...] = jnp.zeros_like(st)

    @pl.when(ph == 0)
    def _():
        gm = g_ref[...]
        acc(0, x_pre(gm))
        acc(1, wn_pre(gm, 1))
        acc(4, dn_pre(gm, 1))

    @pl.when(ph == 1)
    def _():
        gm = g_ref[...]
        acc(2, wn_pre(gm, 2))
        acc(5, dn_pre(gm, 2))

    @pl.when(ph == 2)
    def _():
        gm = g_ref[...]
        acc(3, wn_pre(gm, 3))
        acc(6, dn_pre(gm, 3))

    @pl.when(ph == 3)
    def _():
        gm = g_ref[...]
        xx = jnp.maximum(bn(0, x_pre(gm), mg, mbe), 0.0)
        ds = jax.nn.sigmoid(bn(6, dn_pre(gm, 3), dg3, dbe3))
        xx = xx * ds
        w = jnp.maximum(bn(3, wn_pre(gm, 3), wg3, wbe3), 0.0)
        y = jnp.concatenate(
            [jnp.sum(xx * w[:, :, k:k + 1], axis=1) for k in range(16)],
            axis=1)
        y = lax.dot_general(y, lp[...], (((1,), (0,)), ((), ())),
                            preferred_element_type=jnp.float32,
                            precision=lax.Precision.HIGHEST) + lb[...]
        ys[pl.ds(c * cb, cb), :] = y
        st[14, 0:Co] += jnp.sum(y, axis=0)
        st[15, 0:Co] += jnp.sum(y * y, axis=0)

    @pl.when(ph == 4)
    def _():
        y = ys[pl.ds(c * cb, cb), :]
        m = st[14, 0:Co] / float(BS)
        v = st[15, 0:Co] / float(BS) - m * m
        y = (y - m[None]) / jnp.sqrt(v[None] + EPS) * blg[...] + blbe[...]
        out_ref[...] = jnp.maximum(y, 0.0)


def _prep_params(p):
    def cv(t):
        w, b, g, be = t
        co = w.shape[0]
        return (w.T, b.reshape(1, co), g.reshape(1, co), be.reshape(1, co))

    flat = list(cv((p['mlp_W'], p['mlp_b'], p['mlp_g'], p['mlp_be'])))
    for t in p['wn']:
        flat += list(cv(t))
    for t in p['dn']:
        flat += list(cv(t))
    co = p['lin_W'].shape[0]
    lw = p['lin_W'].reshape(co, co, 16).transpose(0, 2, 1).reshape(co, 16 * co)
    flat += [lw.T, p['lin_b'].reshape(1, co),
             p['bnl_g'].reshape(1, co), p['bnl_be'].reshape(1, co)]
    return flat


def _sa_tail(gm, nx, p, C, Co, pt_lo, dc, nchunk=8):
    bs, kk, d = gm.shape
    cb = bs // nchunk
    cmax = max(Co, 16)
    flat = _prep_params(p)

    def body(*refs):
        _sa_tail_body(bs, kk, C, Co, pt_lo, dc, cmax, refs)

    full = lambda shape: pl.BlockSpec(shape, lambda ph, c: (0,) * len(shape))
    wspecs = [full(a.shape) for a in flat]
    return pl.pallas_call(
        body,
        grid=(5, nchunk),
        in_specs=[pl.BlockSpec((cb, kk, d), lambda ph, c: (c, 0, 0)),
                  pl.BlockSpec((cb, 1, 3), lambda ph, c: (c, 0, 0))]
        + wspecs,
        out_specs=pl.BlockSpec((cb, Co), lambda ph, c: (c, 0)),
        out_shape=jax.ShapeDtypeStruct((bs, Co), jnp.float32),
        scratch_shapes=[pltpu.VMEM((16, cmax), jnp.float32),
                        pltpu.VMEM((bs, Co), jnp.float32)],
        compiler_params=pltpu.CompilerParams(
            vmem_limit_bytes=60 * 1024 * 1024),
    )(gm, nx, *flat)


def _sa3_body(B, N, bw, Co, refs):
    (xyzT_ref, xyz_ref, pts_ref,
     mw, mb, mg, mbe,
     w1, wb1, wg1, wbe1, w2, wb2, wg2, wbe2, w3, wb3, wg3, wbe3,
     d1, db1, dg1, dbe1, d2, db2, dg2, dbe2, d3, db3, dg3, dbe3,
     lp, lb, blg, blbe, out_ref) = refs
    cols = []
    for b in range(B):
        xt = xyzT_ref[b]
        g = lax.dot_general(xt, xyz_ref[b], (((1,), (0,)), ((), ())),
                            preferred_element_type=jnp.float32)
        n2 = jnp.sum(xt * xt, axis=1)
        sq = -2.0 * g + n2[:, None] + n2[None, :]
        dens = jnp.mean(jnp.exp(-sq / (2.0 * bw * bw)) / (2.5 * bw),
                        axis=1, keepdims=True)
        cols.append((1.0 / dens)[None])
    invd = jnp.concatenate(cols, axis=0)

    gx = xyzT_ref[...]
    npts = jnp.concatenate([gx, pts_ref[...]], axis=2)
    xx = jnp.maximum(_bn3(_conv(npts, mw, mb), mg, mbe), 0.0)

    ratio = invd / jnp.max(invd, axis=1, keepdims=True)
    h = jnp.maximum(_bn3(_conv(ratio, d1, db1), dg1, dbe1), 0.0)
    h = jnp.maximum(_bn3(_conv(h, d2, db2), dg2, dbe2), 0.0)
    ds = jax.nn.sigmoid(_bn3(_conv(h, d3, db3), dg3, dbe3))
    xx = xx * ds

    w = jnp.maximum(_bn3(_conv(gx, w1, wb1), wg1, wbe1), 0.0)
    w = jnp.maximum(_bn3(_conv(w, w2, wb2), wg2, wbe2), 0.0)
    w = jnp.maximum(_bn3(_conv(w, w3, wb3), wg3, wbe3), 0.0)

    y = jnp.concatenate(
        [jnp.sum(xx * w[:, :, k:k + 1], axis=1) for k in range(16)], axis=1)
    y = lax.dot_general(y, lp[...], (((1,), (0,)), ((), ())),
                        preferred_element_type=jnp.float32,
                        precision=lax.Precision.HIGHEST) + lb[...]
    m = jnp.mean(y, axis=0)
    v = jnp.mean((y - m) ** 2, axis=0)
    y = (y - m[None]) / jnp.sqrt(v[None] + EPS) * blg[...] + blbe[...]
    out_ref[...] = jnp.maximum(y, 0.0)


def _sa3(xyzT, xyz, pts, p, bw, Co):
    B, N, _ = xyzT.shape
    flat = _prep_params(p)

    def body(*refs):
        _sa3_body(B, N, bw, Co, refs)

    return pl.pallas_call(
        body,
        out_shape=jax.ShapeDtypeStruct((B, Co), jnp.float32),
    )(xyzT, xyz, pts, *flat)


def kernel(pc, params):
    f32 = jnp.float32
    B, _, N1 = pc.shape
    S1, K1 = 512, 32
    S2, K2 = 128, 64

    xyzT1 = jnp.transpose(pc, (0, 2, 1))
    invd1, nx1, idx1 = _geom(pc, xyzT1, S1, K1, 0.1)
    t1 = jnp.concatenate(
        [xyzT1, invd1, jnp.zeros((B, N1, 4), f32)], axis=2).reshape(B * N1, 8)
    g1 = _gather_rows(t1, idx1.reshape(-1)).reshape(B * S1, K1, 8)
    nxp1 = nx1.reshape(B * S1, 1, 3)
    y1 = _sa_tail(g1, nxp1, params['sa1'], C=3, Co=64, pt_lo=0, dc=3,
                  nchunk=16)

    xyz2 = jnp.transpose(nx1, (0, 2, 1))
    invd2, nx2, idx2 = _geom(xyz2, nx1, S2, K2, 0.2)
    pts2 = y1.reshape(B, S1, 64)
    t2 = jnp.concatenate(
        [nx1, pts2, invd2, jnp.zeros((B, S1, 4), f32)],
        axis=2).reshape(B * S1, 72)
    g2 = _gather_rows(t2, idx2.reshape(-1)).reshape(B * S2, K2, 72)
    nxp2 = nx2.reshape(B * S2, 1, 3)
    y2 = _sa_tail(g2, nxp2, params['sa2'], C=64, Co=128, pt_lo=3, dc=67)

    xyz3 = jnp.transpose(nx2, (0, 2, 1))
    pts3 = y2.reshape(B, S2, 128)
    return _sa3(nx2, xyz3, pts3, params['sa3'], 0.4, 256)
